# baseline (speedup 1.0000x reference)
import functools

import jax
import jax.numpy as jnp
from jax import lax
from jax.experimental import pallas as pl
from jax.experimental.pallas import tpu as pltpu
from jax.experimental.pallas import tpu_sc as plsc

_N_EXPERTS = 8
_RANK = 64
_SCALING = 128.0 / 64.0
_BT1 = 1024
_BT2 = 1024
_NEG = float("-inf")


def _sc_routing(p_t):
    E, T = p_t.shape
    info = plsc.get_sparse_core_info()
    nw = info.num_cores * info.num_subcores
    tpw = T // nw
    mesh = plsc.VectorSubcoreMesh(core_axis_name="c", subcore_axis_name="s")

    @functools.partial(
        pl.kernel, mesh=mesh,
        out_type=jax.ShapeDtypeStruct((E, T), jnp.float32),
        scratch_types=[
            pltpu.VMEM((E, tpw), jnp.float32),
            pltpu.VMEM((E, tpw), jnp.float32),
        ],
    )
    def k(p_hbm, w_hbm, p_v, w_v):
        wid = lax.axis_index("s") * info.num_cores + lax.axis_index("c")
        base = wid * tpw
        pltpu.sync_copy(p_hbm.at[:, pl.ds(base, tpw)], p_v)
        for c in range(tpw // 16):
            sl = pl.ds(c * 16, 16)
            v = [p_v[e, sl] for e in range(E)]
            m1 = v[0]
            for e in range(1, E):
                m1 = jnp.maximum(m1, v[e])
            i1 = jnp.zeros((16,), jnp.float32) + float(E)
            for e in reversed(range(E)):
                i1 = jnp.where(v[e] == m1, float(e), i1)
            vm = [jnp.where(i1 == float(e), _NEG, v[e]) for e in range(E)]
            m2 = vm[0]
            for e in range(1, E):
                m2 = jnp.maximum(m2, vm[e])
            i2 = jnp.zeros((16,), jnp.float32) + float(E)
            for e in reversed(range(E)):
                i2 = jnp.where(vm[e] == m2, float(e), i2)
            for e in range(E):
                fe = float(e)
                w_v[e, sl] = jnp.where((i1 == fe) | (i2 == fe), v[e], 0.0)
        pltpu.sync_copy(w_v, w_hbm.at[:, pl.ds(base, tpw)])

    return k(p_t)


def _tc1_body(h_ref, a_ref, u_ref):
    hb = h_ref[...].astype(jnp.bfloat16)
    u = jax.lax.dot_general(hb, a_ref[...], (((1,), (1,)), ((), ())),
                            preferred_element_type=jnp.float32)
    u_ref[...] = u.astype(jnp.bfloat16)


def _tc2_body(w_ref, u_ref, b_ref, s_ref, o_ref):
    wrep = jax.lax.dot_general(w_ref[...], s_ref[...], (((0,), (0,)), ((), ())),
                               preferred_element_type=jnp.float32)
    v = (u_ref[...].astype(jnp.float32) * wrep).astype(jnp.bfloat16)
    o_ref[...] = jax.lax.dot_general(v, b_ref[...], (((1,), (0,)), ((), ())),
                                     preferred_element_type=jnp.float32)


def kernel(h, p_L, A, B):
    T, D = h.shape
    E, R, _ = A.shape
    ER = E * R
    a_cat = A.reshape(ER, D).astype(jnp.bfloat16)
    b_cat = B.transpose(0, 2, 1).reshape(ER, D).astype(jnp.bfloat16)
    sel = _SCALING * jnp.repeat(jnp.eye(E, dtype=jnp.float32), R, axis=1)

    w_t = _sc_routing(p_L.T)

    u = pl.pallas_call(
        _tc1_body,
        grid=(T // _BT1,),
        in_specs=[
            pl.BlockSpec((_BT1, D), lambda i: (i, 0)),
            pl.BlockSpec((ER, D), lambda i: (0, 0)),
        ],
        out_specs=pl.BlockSpec((_BT1, ER), lambda i: (i, 0)),
        out_shape=jax.ShapeDtypeStruct((T, ER), jnp.bfloat16),
    )(h, a_cat)

    return pl.pallas_call(
        _tc2_body,
        grid=(T // _BT2,),
        in_specs=[
            pl.BlockSpec((E, _BT2), lambda i: (0, i)),
            pl.BlockSpec((_BT2, ER), lambda i: (i, 0)),
            pl.BlockSpec((ER, D), lambda i: (0, 0)),
            pl.BlockSpec((E, ER), lambda i: (0, 0)),
        ],
        out_specs=pl.BlockSpec((_BT2, D), lambda i: (i, 0)),
        out_shape=jax.ShapeDtypeStruct((T, D), h.dtype),
    )(w_t, u, b_cat, sel)


# speedup vs baseline: 1.0754x; 1.0754x over previous
import functools

import jax
import jax.numpy as jnp
from jax import lax
from jax.experimental import pallas as pl
from jax.experimental.pallas import tpu as pltpu
from jax.experimental.pallas import tpu_sc as plsc

_NEG = float("-inf")

_N_EXPERTS = 8
_RANK = 64
_SCALING = 128.0 / 64.0
_BT = 1024


def _sc_routing(p_t):
    E, T = p_t.shape
    info = plsc.get_sparse_core_info()
    nw = info.num_cores * info.num_subcores
    tpw = T // nw
    mesh = plsc.VectorSubcoreMesh(core_axis_name="c", subcore_axis_name="s")

    @functools.partial(
        pl.kernel, mesh=mesh,
        out_type=jax.ShapeDtypeStruct((E, T), jnp.float32),
        scratch_types=[
            pltpu.VMEM((E, tpw), jnp.float32),
            pltpu.VMEM((E, tpw), jnp.float32),
        ],
    )
    def k(p_hbm, w_hbm, p_v, w_v):
        wid = lax.axis_index("s") * info.num_cores + lax.axis_index("c")
        base = wid * tpw
        pltpu.sync_copy(p_hbm.at[:, pl.ds(base, tpw)], p_v)
        for c in range(tpw // 16):
            sl = pl.ds(c * 16, 16)
            v = [p_v[e, sl] for e in range(E)]
            m1 = v[0]
            for e in range(1, E):
                m1 = jnp.maximum(m1, v[e])
            i1 = jnp.zeros((16,), jnp.float32) + float(E)
            for e in reversed(range(E)):
                i1 = jnp.where(v[e] == m1, float(e), i1)
            vm = [jnp.where(i1 == float(e), _NEG, v[e]) for e in range(E)]
            m2 = vm[0]
            for e in range(1, E):
                m2 = jnp.maximum(m2, vm[e])
            i2 = jnp.zeros((16,), jnp.float32) + float(E)
            for e in reversed(range(E)):
                i2 = jnp.where(vm[e] == m2, float(e), i2)
            for e in range(E):
                fe = float(e)
                w_v[e, sl] = jnp.where((i1 == fe) | (i2 == fe), v[e], 0.0)
        pltpu.sync_copy(w_v, w_hbm.at[:, pl.ds(base, tpw)])

    return k(p_t)


def _routing_weights_t(p):
    row = jax.lax.broadcasted_iota(jnp.int32, p.shape, 0)
    m1 = jnp.max(p, axis=0, keepdims=True)
    i1 = jnp.min(jnp.where(p == m1, row, _N_EXPERTS), axis=0, keepdims=True)
    sel1 = row == i1
    p2 = jnp.where(sel1, -jnp.inf, p)
    m2 = jnp.max(p2, axis=0, keepdims=True)
    i2 = jnp.min(jnp.where(p2 == m2, row, _N_EXPERTS), axis=0, keepdims=True)
    sel2 = row == i2
    return jnp.where(sel1 | sel2, p, 0.0)


def _body(p_ref, h_ref, a_ref, b_ref, s_ref, o_ref):
    hb = h_ref[...].astype(jnp.bfloat16)
    u = jax.lax.dot_general(hb, a_ref[...], (((1,), (1,)), ((), ())),
                            preferred_element_type=jnp.float32)
    w = p_ref[...]
    wrep = jax.lax.dot_general(w, s_ref[...], (((0,), (0,)), ((), ())),
                               preferred_element_type=jnp.float32)
    v = (u * wrep).astype(jnp.bfloat16)
    o_ref[...] = jax.lax.dot_general(v, b_ref[...], (((1,), (0,)), ((), ())),
                                     preferred_element_type=jnp.float32)


def kernel(h, p_L, A, B):
    T, D = h.shape
    E, R, _ = A.shape
    ER = E * R
    a_cat = A.reshape(ER, D).astype(jnp.bfloat16)
    b_cat = B.transpose(0, 2, 1).reshape(ER, D).astype(jnp.bfloat16)
    sel = _SCALING * jnp.repeat(jnp.eye(E, dtype=jnp.float32), R, axis=1)
    grid = (T // _BT,)
    return pl.pallas_call(
        _body,
        grid=grid,
        in_specs=[
            pl.BlockSpec((E, _BT), lambda i: (0, i)),
            pl.BlockSpec((_BT, D), lambda i: (i, 0)),
            pl.BlockSpec((ER, D), lambda i: (0, 0)),
            pl.BlockSpec((ER, D), lambda i: (0, 0)),
            pl.BlockSpec((E, ER), lambda i: (0, 0)),
        ],
        out_specs=pl.BlockSpec((_BT, D), lambda i: (i, 0)),
        out_shape=jax.ShapeDtypeStruct((T, D), h.dtype),
    )(_sc_routing(p_L.T), h, a_cat, b_cat, sel)


# speedup vs baseline: 1.3888x; 1.2914x over previous
import jax
import jax.numpy as jnp
from jax.experimental import pallas as pl
from jax.experimental.pallas import tpu as pltpu

_N_EXPERTS = 8
_RANK = 64
_SCALING = 128.0 / 64.0
_BT = 1024


def _routing_weights_t(p):
    row = jax.lax.broadcasted_iota(jnp.int32, p.shape, 0)
    m1 = jnp.max(p, axis=0, keepdims=True)
    i1 = jnp.min(jnp.where(p == m1, row, _N_EXPERTS), axis=0, keepdims=True)
    sel1 = row == i1
    p2 = jnp.where(sel1, -jnp.inf, p)
    m2 = jnp.max(p2, axis=0, keepdims=True)
    i2 = jnp.min(jnp.where(p2 == m2, row, _N_EXPERTS), axis=0, keepdims=True)
    sel2 = row == i2
    return jnp.where(sel1 | sel2, p, 0.0)


def _body(p_ref, h_ref, a_ref, b_ref, s_ref, o_ref):
    hb = h_ref[...].astype(jnp.bfloat16)
    u = jax.lax.dot_general(hb, a_ref[...], (((1,), (1,)), ((), ())),
                            preferred_element_type=jnp.float32)
    w = _routing_weights_t(p_ref[...])
    wrep = jax.lax.dot_general(w, s_ref[...], (((0,), (0,)), ((), ())),
                               preferred_element_type=jnp.float32)
    v = (u * wrep).astype(jnp.bfloat16)
    o_ref[...] = jax.lax.dot_general(v, b_ref[...], (((1,), (0,)), ((), ())),
                                     preferred_element_type=jnp.float32)


def kernel(h, p_L, A, B):
    T, D = h.shape
    E, R, _ = A.shape
    ER = E * R
    a_cat = A.reshape(ER, D).astype(jnp.bfloat16)
    b_cat = B.transpose(0, 2, 1).reshape(ER, D).astype(jnp.bfloat16)
    sel = _SCALING * jnp.repeat(jnp.eye(E, dtype=jnp.float32), R, axis=1)
    grid = (T // _BT,)
    return pl.pallas_call(
        _body,
        grid=grid,
        in_specs=[
            pl.BlockSpec((E, _BT), lambda i: (0, i)),
            pl.BlockSpec((_BT, D), lambda i: (i, 0)),
            pl.BlockSpec((ER, D), lambda i: (0, 0)),
            pl.BlockSpec((ER, D), lambda i: (0, 0)),
            pl.BlockSpec((E, ER), lambda i: (0, 0)),
        ],
        out_specs=pl.BlockSpec((_BT, D), lambda i: (i, 0)),
        out_shape=jax.ShapeDtypeStruct((T, D), h.dtype),
    )(p_L.T, h, a_cat, b_cat, sel)


# speedup vs baseline: 1.4043x; 1.0112x over previous
import jax
import jax.numpy as jnp
from jax.experimental import pallas as pl
from jax.experimental.pallas import tpu as pltpu

_N_EXPERTS = 8
_RANK = 64
_SCALING = 128.0 / 64.0
_BT = 1024


def _routing_weights_t(p):
    row = jax.lax.broadcasted_iota(jnp.int32, p.shape, 0)
    m1 = jnp.max(p, axis=0, keepdims=True)
    i1 = jnp.min(jnp.where(p == m1, row, _N_EXPERTS), axis=0, keepdims=True)
    sel1 = row == i1
    p2 = jnp.where(sel1, -jnp.inf, p)
    m2 = jnp.max(p2, axis=0, keepdims=True)
    i2 = jnp.min(jnp.where(p2 == m2, row, _N_EXPERTS), axis=0, keepdims=True)
    sel2 = row == i2
    return jnp.where(sel1 | sel2, p, 0.0)


def _body(p_ref, h_ref, a_ref, b_ref, s_ref, o_ref, a_bf):
    @pl.when(pl.program_id(0) == 0)
    def _cast_a():
        a_bf[...] = a_ref[...].astype(jnp.bfloat16)

    hb = h_ref[...].astype(jnp.bfloat16)
    u = jax.lax.dot_general(hb, a_bf[...], (((1,), (1,)), ((), ())),
                            preferred_element_type=jnp.float32)
    w = _routing_weights_t(p_ref[...])
    wrep = jax.lax.dot_general(w, s_ref[...], (((0,), (0,)), ((), ())),
                               preferred_element_type=jnp.float32)
    v = (u * wrep).astype(jnp.bfloat16)
    o_ref[...] = jax.lax.dot_general(v, b_ref[...], (((1,), (0,)), ((), ())),
                                     preferred_element_type=jnp.float32)


def kernel(h, p_L, A, B):
    T, D = h.shape
    E, R, _ = A.shape
    ER = E * R
    a_cat = A.reshape(ER, D)
    b_cat = B.transpose(0, 2, 1).reshape(ER, D).astype(jnp.bfloat16)
    sel = _SCALING * jnp.repeat(jnp.eye(E, dtype=jnp.float32), R, axis=1)
    grid = (T // _BT,)
    return pl.pallas_call(
        _body,
        grid=grid,
        in_specs=[
            pl.BlockSpec((E, _BT), lambda i: (0, i)),
            pl.BlockSpec((_BT, D), lambda i: (i, 0)),
            pl.BlockSpec((ER, D), lambda i: (0, 0)),
            pl.BlockSpec((ER, D), lambda i: (0, 0)),
            pl.BlockSpec((E, ER), lambda i: (0, 0)),
        ],
        out_specs=pl.BlockSpec((_BT, D), lambda i: (i, 0)),
        out_shape=jax.ShapeDtypeStruct((T, D), h.dtype),
        scratch_shapes=[pltpu.VMEM((ER, D), jnp.bfloat16)],
    )(p_L.T, h, a_cat, b_cat, sel)
